# baseline (speedup 1.0000x reference)
import functools

import jax
import jax.numpy as jnp
from jax import lax
from jax.experimental import pallas as pl
from jax.experimental.pallas import tpu as pltpu
from jax.experimental.pallas import tpu_sc as plsc

B = 4096
D = 2048
S = 4096

NC = 2
NS = 16
NW = NC * NS
R_CHUNK = 8

_MIN32 = -2147483648
A_PAD = S + 16
L_PAD = D + 16
M_PAD = D + 16
BMASK = 8191


def _sc_body(rows_per_w, x_hbm, h1_hbm, h2_hbm, h3_hbm, s1_hbm, s2_hbm,
             s3_hbm, out_hbm,
             c1, c2, c3, a1, a2, a3,
             kl1, kl2, kl3, ml,
             xb0, xb1, ob0, ob1,
             in_sem0, in_sem1, out_sem0, out_sem1):
    wid = lax.axis_index("c") * NS + lax.axis_index("s")
    row0 = wid * rows_per_w
    n_chunk = rows_per_w // R_CHUNK

    zeros16 = jnp.zeros((16,), jnp.float32)
    ones16 = jnp.ones((16,), jnp.float32)
    iota16 = lax.iota(jnp.int32, 16)

    pltpu.sync_copy(h1_hbm, c1)
    pltpu.sync_copy(h2_hbm, c2)
    pltpu.sync_copy(h3_hbm, c3)
    for s_hbm, cref in ((s1_hbm, c1), (s2_hbm, c2), (s3_hbm, c3)):
        pltpu.sync_copy(s_hbm, xb0.at[0])

        @plsc.parallel_loop(0, D // 16, unroll=8)
        def pack(j, cref=cref):
            dsl = pl.ds(j * 16, 16)
            sbit = jnp.where(xb0[0, dsl] < 0.0,
                             jnp.full((16,), _MIN32, jnp.int32),
                             jnp.zeros((16,), jnp.int32))
            dv = jnp.left_shift(j * 16 + iota16, 13)
            cref[dsl] = jnp.bitwise_or(jnp.bitwise_or(cref[dsl], sbit), dv)

    @plsc.parallel_loop(0, A_PAD // 16, unroll=8)
    def zero_acc(k):
        dsl = pl.ds(k * 16, 16)
        a1[dsl] = zeros16
        a2[dsl] = zeros16
        a3[dsl] = zeros16

    for cref, accref in ((c1, a1), (c2, a2), (c3, a3)):
        @plsc.parallel_loop(0, D // 16, unroll=8)
        def touch(j, cref=cref, accref=accref):
            dsl = pl.ds(j * 16, 16)
            idx = jnp.bitwise_and(cref[dsl], S - 1)
            plsc.store_scatter(accref, [idx], ones16)

    def build_list(cref, klref):
        def step(j, off):
            dsl = pl.ds(j * 16, 16)
            cv = cref[dsl]
            idx = jnp.bitwise_and(cv, S - 1)
            t1 = plsc.load_gather(a1, [idx])
            t2 = plsc.load_gather(a2, [idx])
            t3 = plsc.load_gather(a3, [idx])
            keep = (t1 * t2 * t3) > 0.5
            plsc.store_compressed(klref.at[pl.ds(off, 16)], cv, mask=keep)
            cnt = jnp.max(plsc.all_reduce_population_count(keep))
            return off + cnt

        nk = lax.fori_loop(0, D // 16, step, jnp.int32(0))
        klref[pl.ds(nk, 16)] = jnp.full((16,), S, jnp.int32)
        return (nk + 15) >> 4

    nt1 = build_list(c1, kl1)
    nt2 = build_list(c2, kl2)
    nt3 = build_list(c3, kl3)

    def mstep(k, off):
        dsl = pl.ds(k * 16, 16)
        keep = (a1[dsl] * a2[dsl] * a3[dsl]) > 0.5
        plsc.store_compressed(ml.at[pl.ds(off, 16)], k * 16 + iota16,
                              mask=keep)
        cnt = jnp.max(plsc.all_reduce_population_count(keep))
        return off + cnt

    nm = lax.fori_loop(0, S // 16, mstep, jnp.int32(0))
    ml[pl.ds(nm, 16)] = jnp.full((16,), S, jnp.int32)
    ntm = (nm + 15) >> 4

    for cref, accref in ((c1, a1), (c2, a2), (c3, a3)):
        @plsc.parallel_loop(0, D // 16, unroll=8)
        def untouch(j, cref=cref, accref=accref):
            dsl = pl.ds(j * 16, 16)
            idx = jnp.bitwise_and(cref[dsl], S - 1)
            plsc.store_scatter(accref, [idx], zeros16)

    for ob in (ob0, ob1):
        def zrow(r, _, ob=ob):
            @plsc.parallel_loop(0, S // 16, unroll=8)
            def zero_ob(k):
                ob[r, pl.ds(k * 16, 16)] = zeros16
            return 0

        lax.fori_loop(0, R_CHUNK, zrow, 0)

    def compute(xb, ob):
        def row_body(r, _):
            rvec = jnp.full((16,), r, jnp.int32)
            for klref, accref, nt in ((kl1, a1, nt1), (kl2, a2, nt2),
                                      (kl3, a3, nt3)):
                @plsc.parallel_loop(0, nt, unroll=1)
                def scat(j, klref=klref, accref=accref, rvec=rvec):
                    dsl = pl.ds(j * 16, 16)
                    cv = klref[dsl]
                    dv = jnp.bitwise_and(lax.shift_right_logical(cv, 13),
                                         D - 1)
                    xv = plsc.load_gather(xb, [rvec, dv])
                    idx = jnp.bitwise_and(cv, BMASK)
                    val = xv * jnp.where(cv < 0,
                                         jnp.full((16,), -1.0, jnp.float32),
                                         jnp.full((16,), 1.0, jnp.float32))
                    plsc.addupdate_scatter(accref, [idx], val)

            @plsc.parallel_loop(0, ntm, unroll=1)
            def prod(j, rvec=rvec):
                dsl = pl.ds(j * 16, 16)
                bv = ml[dsl]
                p = (plsc.load_gather(a1, [bv])
                     * plsc.load_gather(a2, [bv])
                     * plsc.load_gather(a3, [bv]))
                plsc.store_scatter(ob, [rvec, bv], p, mask=bv < S)
                plsc.store_scatter(a1, [bv], zeros16)
                plsc.store_scatter(a2, [bv], zeros16)
                plsc.store_scatter(a3, [bv], zeros16)
            return 0

        lax.fori_loop(0, R_CHUNK, row_body, 0)

    def clear_ob(ob):
        def row_body(r, _):
            rvec = jnp.full((16,), r, jnp.int32)

            @plsc.parallel_loop(0, ntm, unroll=1)
            def zb(j, rvec=rvec):
                bv = ml[pl.ds(j * 16, 16)]
                plsc.store_scatter(ob, [rvec, bv], zeros16, mask=bv < S)
            return 0

        lax.fori_loop(0, R_CHUNK, row_body, 0)

    bufs = ((xb0, ob0, in_sem0, out_sem0), (xb1, ob1, in_sem1, out_sem1))

    def in_copy(g, xb, isem):
        return pltpu.make_async_copy(
            x_hbm.at[pl.ds(row0 + g * R_CHUNK, R_CHUNK)], xb, isem)

    def out_copy(g, ob, osem):
        return pltpu.make_async_copy(
            ob, out_hbm.at[pl.ds(row0 + g * R_CHUNK, R_CHUNK)], osem)

    in_copy(0, xb0, in_sem0).start()
    in_copy(1, xb1, in_sem1).start()

    def pair_body(p, _):
        for b in range(2):
            xb, ob, isem, osem = bufs[b]
            g = 2 * p + b
            in_copy(g, xb, isem).wait()

            @pl.when(p > 0)
            def _drain_out():
                out_copy(g - 2, ob, osem).wait()
                clear_ob(ob)

            compute(xb, ob)
            out_copy(g, ob, osem).start()

            @pl.when(g + 2 < n_chunk)
            def _prefetch():
                in_copy(g + 2, xb, isem).start()
        return 0

    lax.fori_loop(0, n_chunk // 2, pair_body, 0)
    out_copy(n_chunk - 2, ob0, out_sem0).wait()
    out_copy(n_chunk - 1, ob1, out_sem1).wait()


def _tensor_sketch_sc(x, hash1, hash2, hash3, sign1, sign2, sign3):
    rows = x.shape[0]
    rows_per_w = rows // NW
    mesh = plsc.VectorSubcoreMesh(core_axis_name="c", subcore_axis_name="s")
    k = functools.partial(
        pl.kernel, mesh=mesh,
        out_type=jax.ShapeDtypeStruct((rows, S), jnp.float32),
        compiler_params=pltpu.CompilerParams(needs_layout_passes=False),
        scratch_types=[
            pltpu.VMEM((D,), jnp.int32),
            pltpu.VMEM((D,), jnp.int32),
            pltpu.VMEM((D,), jnp.int32),
            pltpu.VMEM((A_PAD,), jnp.float32),
            pltpu.VMEM((A_PAD,), jnp.float32),
            pltpu.VMEM((A_PAD,), jnp.float32),
            pltpu.VMEM((L_PAD,), jnp.int32),
            pltpu.VMEM((L_PAD,), jnp.int32),
            pltpu.VMEM((L_PAD,), jnp.int32),
            pltpu.VMEM((M_PAD,), jnp.int32),
            pltpu.VMEM((R_CHUNK, D), jnp.float32),
            pltpu.VMEM((R_CHUNK, D), jnp.float32),
            pltpu.VMEM((R_CHUNK, S), jnp.float32),
            pltpu.VMEM((R_CHUNK, S), jnp.float32),
            pltpu.SemaphoreType.DMA,
            pltpu.SemaphoreType.DMA,
            pltpu.SemaphoreType.DMA,
            pltpu.SemaphoreType.DMA,
        ],
    )(functools.partial(_sc_body, rows_per_w))
    return k(x, hash1, hash2, hash3, sign1, sign2, sign3)


@jax.jit
def kernel(x, sign1, sign2, sign3, hash1, hash2, hash3):
    return _tensor_sketch_sc(x, hash1, hash2, hash3, sign1, sign2, sign3)


# speedup vs baseline: 1.2294x; 1.2294x over previous
import functools

import jax
import jax.numpy as jnp
from jax import lax
from jax.experimental import pallas as pl
from jax.experimental.pallas import tpu as pltpu
from jax.experimental.pallas import tpu_sc as plsc

B = 4096
D = 2048
S = 4096

NC = 2
NS = 16
NW = NC * NS
R_CHUNK = 8

_MIN32 = -2147483648
A_PAD = S + 16
L_PAD = D + 16
M_PAD = D + 16
BMASK = 8191


def _sc_body(rows_per_w, x_hbm, h1_hbm, h2_hbm, h3_hbm, s1_hbm, s2_hbm,
             s3_hbm, out_hbm,
             c1, c2, c3, a1, a2, a3,
             kl1, kl2, kl3, ml,
             xb0, xb1, ob0, ob1,
             in_sem0, in_sem1, out_sem0, out_sem1):
    wid = lax.axis_index("c") * NS + lax.axis_index("s")
    row0 = wid * rows_per_w
    n_chunk = rows_per_w // R_CHUNK

    def _early_in(g, xb, isem):
        return pltpu.make_async_copy(
            x_hbm.at[pl.ds(row0 + g * R_CHUNK, R_CHUNK)], xb, isem)

    _early_in(0, xb0, in_sem0).start()
    _early_in(1, xb1, in_sem1).start()

    zeros16 = jnp.zeros((16,), jnp.float32)
    ones16 = jnp.ones((16,), jnp.float32)
    iota16 = lax.iota(jnp.int32, 16)

    pltpu.sync_copy(h1_hbm, c1)
    pltpu.sync_copy(h2_hbm, c2)
    pltpu.sync_copy(h3_hbm, c3)
    for s_hbm, cref in ((s1_hbm, c1), (s2_hbm, c2), (s3_hbm, c3)):
        pltpu.sync_copy(s_hbm, ob0.at[0, pl.ds(0, D)])

        @plsc.parallel_loop(0, D // 16, unroll=8)
        def pack(j, cref=cref):
            dsl = pl.ds(j * 16, 16)
            sbit = jnp.where(ob0[0, dsl] < 0.0,
                             jnp.full((16,), _MIN32, jnp.int32),
                             jnp.zeros((16,), jnp.int32))
            dv = jnp.left_shift(j * 16 + iota16, 13)
            cref[dsl] = jnp.bitwise_or(jnp.bitwise_or(cref[dsl], sbit), dv)

    @plsc.parallel_loop(0, A_PAD // 16, unroll=8)
    def zero_acc(k):
        dsl = pl.ds(k * 16, 16)
        a1[dsl] = zeros16
        a2[dsl] = zeros16
        a3[dsl] = zeros16

    for cref, accref in ((c1, a1), (c2, a2), (c3, a3)):
        @plsc.parallel_loop(0, D // 16, unroll=8)
        def touch(j, cref=cref, accref=accref):
            dsl = pl.ds(j * 16, 16)
            idx = jnp.bitwise_and(cref[dsl], S - 1)
            plsc.store_scatter(accref, [idx], ones16)

    def build_list(cref, klref):
        def step(j, off):
            dsl = pl.ds(j * 16, 16)
            cv = cref[dsl]
            idx = jnp.bitwise_and(cv, S - 1)
            t1 = plsc.load_gather(a1, [idx])
            t2 = plsc.load_gather(a2, [idx])
            t3 = plsc.load_gather(a3, [idx])
            keep = (t1 * t2 * t3) > 0.5
            plsc.store_compressed(klref.at[pl.ds(off, 16)], cv, mask=keep)
            cnt = jnp.max(plsc.all_reduce_population_count(keep))
            return off + cnt

        nk = lax.fori_loop(0, D // 16, step, jnp.int32(0))
        klref[pl.ds(nk, 16)] = jnp.full((16,), S, jnp.int32)
        return (nk + 15) >> 4

    nt1 = build_list(c1, kl1)
    nt2 = build_list(c2, kl2)
    nt3 = build_list(c3, kl3)

    def mstep(k, off):
        dsl = pl.ds(k * 16, 16)
        keep = (a1[dsl] * a2[dsl] * a3[dsl]) > 0.5
        plsc.store_compressed(ml.at[pl.ds(off, 16)], k * 16 + iota16,
                              mask=keep)
        cnt = jnp.max(plsc.all_reduce_population_count(keep))
        return off + cnt

    nm = lax.fori_loop(0, S // 16, mstep, jnp.int32(0))
    ml[pl.ds(nm, 16)] = jnp.full((16,), S, jnp.int32)
    ntm = (nm + 15) >> 4

    for cref, accref in ((c1, a1), (c2, a2), (c3, a3)):
        @plsc.parallel_loop(0, D // 16, unroll=8)
        def untouch(j, cref=cref, accref=accref):
            dsl = pl.ds(j * 16, 16)
            idx = jnp.bitwise_and(cref[dsl], S - 1)
            plsc.store_scatter(accref, [idx], zeros16)

    for ob in (ob0, ob1):
        def zrow(r, _, ob=ob):
            @plsc.parallel_loop(0, S // 16, unroll=8)
            def zero_ob(k):
                ob[r, pl.ds(k * 16, 16)] = zeros16
            return 0

        lax.fori_loop(0, R_CHUNK, zrow, 0)

    def compute(xb, ob):
        def row_body(r, _):
            rvec = jnp.full((16,), r, jnp.int32)
            for klref, accref, nt in ((kl1, a1, nt1), (kl2, a2, nt2),
                                      (kl3, a3, nt3)):
                @plsc.parallel_loop(0, nt, unroll=2)
                def scat(j, klref=klref, accref=accref, rvec=rvec):
                    dsl = pl.ds(j * 16, 16)
                    cv = klref[dsl]
                    dv = jnp.bitwise_and(lax.shift_right_logical(cv, 13),
                                         D - 1)
                    xv = plsc.load_gather(xb, [rvec, dv])
                    idx = jnp.bitwise_and(cv, BMASK)
                    val = xv * jnp.where(cv < 0,
                                         jnp.full((16,), -1.0, jnp.float32),
                                         jnp.full((16,), 1.0, jnp.float32))
                    plsc.addupdate_scatter(accref, [idx], val)

            @plsc.parallel_loop(0, ntm, unroll=2)
            def prod(j, rvec=rvec):
                dsl = pl.ds(j * 16, 16)
                bv = ml[dsl]
                p = (plsc.load_gather(a1, [bv])
                     * plsc.load_gather(a2, [bv])
                     * plsc.load_gather(a3, [bv]))
                plsc.store_scatter(ob, [rvec, bv], p, mask=bv < S)
                plsc.store_scatter(a1, [bv], zeros16)
                plsc.store_scatter(a2, [bv], zeros16)
                plsc.store_scatter(a3, [bv], zeros16)
            return 0

        lax.fori_loop(0, R_CHUNK, row_body, 0)

    bufs = ((xb0, ob0, in_sem0, out_sem0), (xb1, ob1, in_sem1, out_sem1))

    def in_copy(g, xb, isem):
        return pltpu.make_async_copy(
            x_hbm.at[pl.ds(row0 + g * R_CHUNK, R_CHUNK)], xb, isem)

    def out_copy(g, ob, osem):
        return pltpu.make_async_copy(
            ob, out_hbm.at[pl.ds(row0 + g * R_CHUNK, R_CHUNK)], osem)


    def pair_body(p, _):
        for b in range(2):
            xb, ob, isem, osem = bufs[b]
            g = 2 * p + b
            in_copy(g, xb, isem).wait()

            @pl.when(p > 0)
            def _drain_out():
                out_copy(g - 2, ob, osem).wait()

            compute(xb, ob)
            out_copy(g, ob, osem).start()

            @pl.when(g + 2 < n_chunk)
            def _prefetch():
                in_copy(g + 2, xb, isem).start()
        return 0

    lax.fori_loop(0, n_chunk // 2, pair_body, 0)
    out_copy(n_chunk - 2, ob0, out_sem0).wait()
    out_copy(n_chunk - 1, ob1, out_sem1).wait()


def _tensor_sketch_sc(x, hash1, hash2, hash3, sign1, sign2, sign3):
    rows = x.shape[0]
    rows_per_w = rows // NW
    mesh = plsc.VectorSubcoreMesh(core_axis_name="c", subcore_axis_name="s")
    k = functools.partial(
        pl.kernel, mesh=mesh,
        out_type=jax.ShapeDtypeStruct((rows, S), jnp.float32),
        compiler_params=pltpu.CompilerParams(needs_layout_passes=False),
        scratch_types=[
            pltpu.VMEM((D,), jnp.int32),
            pltpu.VMEM((D,), jnp.int32),
            pltpu.VMEM((D,), jnp.int32),
            pltpu.VMEM((A_PAD,), jnp.float32),
            pltpu.VMEM((A_PAD,), jnp.float32),
            pltpu.VMEM((A_PAD,), jnp.float32),
            pltpu.VMEM((L_PAD,), jnp.int32),
            pltpu.VMEM((L_PAD,), jnp.int32),
            pltpu.VMEM((L_PAD,), jnp.int32),
            pltpu.VMEM((M_PAD,), jnp.int32),
            pltpu.VMEM((R_CHUNK, D), jnp.float32),
            pltpu.VMEM((R_CHUNK, D), jnp.float32),
            pltpu.VMEM((R_CHUNK, S), jnp.float32),
            pltpu.VMEM((R_CHUNK, S), jnp.float32),
            pltpu.SemaphoreType.DMA,
            pltpu.SemaphoreType.DMA,
            pltpu.SemaphoreType.DMA,
            pltpu.SemaphoreType.DMA,
        ],
    )(functools.partial(_sc_body, rows_per_w))
    return k(x, hash1, hash2, hash3, sign1, sign2, sign3)


@jax.jit
def kernel(x, sign1, sign2, sign3, hash1, hash2, hash3):
    return _tensor_sketch_sc(x, hash1, hash2, hash3, sign1, sign2, sign3)
